# Initial kernel scaffold; baseline (speedup 1.0000x reference)
import jax
import jax.numpy as jnp
from jax.experimental import pallas as pl


def kernel(x, hand_table, action_table):
    raise NotImplementedError("write your pallas kernel here")


# speedup vs baseline: 2.2198x; 2.2198x over previous
import functools

import jax
import jax.numpy as jnp
from jax import lax
from jax.experimental import pallas as pl
from jax.experimental.pallas import tpu as pltpu
from jax.experimental.pallas import tpu_sc as plsc

B = 16384
D = 128
NC = 2
NS = 16
NW = NC * NS
BPW = B // NW
NCHUNK = 4
CHUNK = BPW // NCHUNK
L = 16


def _fused_body(hand_ref, act_ref, out_ref):
    r = lax.broadcasted_iota(jnp.int32, (32, 1), 0)
    hsel = (r // 6 == lax.broadcasted_iota(jnp.int32, (32, 5), 1)).astype(jnp.float32)
    asel = (r % 6 == lax.broadcasted_iota(jnp.int32, (32, 6), 1)).astype(jnp.float32)
    hand = jnp.dot(hsel, hand_ref[...], preferred_element_type=jnp.float32,
                   precision=lax.Precision.HIGHEST)
    act = jnp.dot(asel, act_ref[...], preferred_element_type=jnp.float32,
                  precision=lax.Precision.HIGHEST)
    pad = jnp.zeros((32, 1), jnp.float32)
    out_ref[...] = jnp.concatenate([hand, act, pad], axis=1)


_mesh = plsc.VectorSubcoreMesh(core_axis_name="c", subcore_axis_name="s")


@functools.partial(
    pl.kernel,
    mesh=_mesh,
    out_type=jax.ShapeDtypeStruct((B, D), jnp.float32),
    compiler_params=pltpu.CompilerParams(needs_layout_passes=False),
    scratch_types=[
        pltpu.VMEM((BPW * 3,), jnp.int32),
        pltpu.VMEM((CHUNK,), jnp.int32),
        pltpu.VMEM((CHUNK,), jnp.int32),
        pltpu.VMEM((CHUNK,), jnp.int32),
        pltpu.VMEM((CHUNK,), jnp.int32),
        pltpu.VMEM((BPW,), jnp.float32),
        pltpu.VMEM((BPW, D), jnp.float32),
        pltpu.SemaphoreType.DMA,
    ],
)
def _gather_kernel(fused_hbm, x_hbm, out_hbm,
                   x_v, i0, i1, i2, i3, bets_v, rows_v, sem):
    wid = lax.axis_index("s") * NC + lax.axis_index("c")
    base = wid * BPW
    pltpu.sync_copy(x_hbm.at[pl.ds(base * 3, BPW * 3)], x_v)
    lanes = lax.iota(jnp.int32, L)
    idx_bufs = [i0, i1, i2, i3]
    for i in range(BPW // L):
        rows = lanes + (i * L)
        flat = rows * 3
        c0 = plsc.load_gather(x_v, [flat])
        c1 = plsc.load_gather(x_v, [flat + 1])
        c2 = plsc.load_gather(x_v, [flat + 2])
        idx_bufs[i // 8][pl.ds((i % 8) * L, L)] = c0 * 6 + c1
        bets_v[pl.ds(i * L, L)] = c2.astype(jnp.float32)
    for j in range(NCHUNK):
        pltpu.async_copy(fused_hbm.at[idx_bufs[j]],
                         rows_v.at[pl.ds(j * CHUNK, CHUNK)], sem).wait()
    col_last = jnp.full((L,), D - 1, jnp.int32)
    for i in range(BPW // L):
        rows = lanes + (i * L)
        plsc.store_scatter(rows_v, [rows, col_last], bets_v[pl.ds(i * L, L)])
    pltpu.sync_copy(rows_v, out_hbm.at[pl.ds(base, BPW)])


def kernel(x, hand_table, action_table):
    x32 = x.astype(jnp.int32)
    fused = pl.pallas_call(
        _fused_body,
        out_shape=jax.ShapeDtypeStruct((32, D), jnp.float32),
    )(hand_table, action_table)
    return _gather_kernel(fused, x32.reshape(-1))
